# baseline (device time: 38930 ns/iter reference)
import functools

import jax
import jax.numpy as jnp
from jax import lax
from jax.experimental import pallas as pl
from jax.experimental.pallas import tpu as pltpu

N_DEV = 32
LOG2 = 5


def kernel(t, W):
    m, _ = t.shape
    _, n = W.shape

    def body(t_ref, w_ref, out_ref, acc_ref, send_ref, recv_ref,
             send_sems, recv_sems):
        my_i = lax.axis_index("i")

        barrier = pltpu.get_barrier_semaphore()
        for r in range(LOG2):
            partner = my_i ^ (1 << r)
            pl.semaphore_signal(
                barrier, inc=1,
                device_id=(partner,), device_id_type=pl.DeviceIdType.MESH,
            )
        pl.semaphore_wait(barrier, LOG2)

        acc_ref[...] = jnp.dot(
            t_ref[...].astype(jnp.bfloat16),
            w_ref[...].astype(jnp.bfloat16),
            preferred_element_type=jnp.float32,
        )

        for r in range(LOG2):
            partner = my_i ^ (1 << r)
            send_ref[...] = acc_ref[...].astype(jnp.bfloat16)
            rdma = pltpu.make_async_remote_copy(
                src_ref=send_ref,
                dst_ref=recv_ref.at[r],
                send_sem=send_sems.at[r],
                recv_sem=recv_sems.at[r],
                device_id=(partner,),
                device_id_type=pl.DeviceIdType.MESH,
            )
            rdma.start()
            rdma.wait()
            acc_ref[...] += recv_ref[r].astype(jnp.float32)

        out_ref[...] = acc_ref[...]

        @functools.partial(
            pl.run_scoped, exit_sem=pltpu.SemaphoreType.REGULAR
        )
        def _(exit_sem):
            for r in range(LOG2):
                partner = my_i ^ (1 << r)
                pl.semaphore_signal(
                    exit_sem, inc=1,
                    device_id=(partner,), device_id_type=pl.DeviceIdType.MESH,
                )
            pl.semaphore_wait(exit_sem, LOG2)

    return pl.pallas_call(
        body,
        out_shape=jax.ShapeDtypeStruct((m, n), jnp.float32),
        in_specs=[
            pl.BlockSpec(memory_space=pltpu.VMEM),
            pl.BlockSpec(memory_space=pltpu.VMEM),
        ],
        out_specs=pl.BlockSpec(memory_space=pltpu.VMEM),
        scratch_shapes=[
            pltpu.VMEM((m, n), jnp.float32),
            pltpu.VMEM((m, n), jnp.bfloat16),
            pltpu.VMEM((LOG2, m, n), jnp.bfloat16),
            pltpu.SemaphoreType.DMA((LOG2,)),
            pltpu.SemaphoreType.DMA((LOG2,)),
        ],
        compiler_params=pltpu.CompilerParams(collective_id=0),
    )(t, W)


# device time: 29588 ns/iter; 1.3157x vs baseline; 1.3157x over previous
import functools

import jax
import jax.numpy as jnp
from jax import lax
from jax.experimental import pallas as pl
from jax.experimental.pallas import tpu as pltpu

N_DEV = 32
SLAB = 16


def kernel(t, W):
    m, _ = t.shape
    _, n = W.shape

    def body(t_ref, w_ref, out_ref, stage_ref, p1recv_ref,
             p1_send_sems, p1_recv_sems, p2_send_sems, p2_recv_sems):
        my_i = lax.axis_index("i")

        barrier = pltpu.get_barrier_semaphore()
        for d in range(1, N_DEV):
            peer = (my_i + d) % N_DEV
            pl.semaphore_signal(
                barrier, inc=1,
                device_id=(peer,), device_id_type=pl.DeviceIdType.MESH,
            )
        pl.semaphore_wait(barrier, N_DEV - 1)

        stage_ref[...] = t_ref[...].astype(jnp.bfloat16)

        p1_sends = []
        for d in range(1, N_DEV):
            tgt = (my_i + d) % N_DEV
            rdma = pltpu.make_async_remote_copy(
                src_ref=stage_ref.at[pl.ds(tgt * SLAB, SLAB)],
                dst_ref=p1recv_ref.at[my_i],
                send_sem=p1_send_sems.at[tgt],
                recv_sem=p1_recv_sems.at[my_i],
                device_id=(tgt,),
                device_id_type=pl.DeviceIdType.MESH,
            )
            rdma.start()
            p1_sends.append(rdma)

        p1recv_ref[pl.ds(my_i, 1)] = stage_ref[pl.ds(my_i * SLAB, SLAB), :][None]
        for d in range(1, N_DEV):
            src_dev = (my_i + d) % N_DEV
            recv = pltpu.make_async_remote_copy(
                src_ref=stage_ref.at[pl.ds(0, SLAB)],
                dst_ref=p1recv_ref.at[src_dev],
                send_sem=p1_send_sems.at[my_i],
                recv_sem=p1_recv_sems.at[src_dev],
                device_id=(my_i,),
                device_id_type=pl.DeviceIdType.MESH,
            )
            recv.wait_recv()

        ssum = jnp.sum(p1recv_ref[...].astype(jnp.float32), axis=0)
        res = jnp.dot(
            ssum.astype(jnp.bfloat16),
            w_ref[...].astype(jnp.bfloat16),
            preferred_element_type=jnp.float32,
        )
        out_ref[pl.ds(my_i * SLAB, SLAB), :] = res

        for rdma in p1_sends:
            rdma.wait_send()

        p2_sends = []
        for d in range(1, N_DEV):
            tgt = (my_i + d) % N_DEV
            rdma = pltpu.make_async_remote_copy(
                src_ref=out_ref.at[pl.ds(my_i * SLAB, SLAB)],
                dst_ref=out_ref.at[pl.ds(my_i * SLAB, SLAB)],
                send_sem=p2_send_sems.at[tgt],
                recv_sem=p2_recv_sems.at[my_i],
                device_id=(tgt,),
                device_id_type=pl.DeviceIdType.MESH,
            )
            rdma.start()
            p2_sends.append(rdma)

        for d in range(1, N_DEV):
            src_dev = (my_i + d) % N_DEV
            recv = pltpu.make_async_remote_copy(
                src_ref=out_ref.at[pl.ds(0, SLAB)],
                dst_ref=out_ref.at[pl.ds(src_dev * SLAB, SLAB)],
                send_sem=p2_send_sems.at[my_i],
                recv_sem=p2_recv_sems.at[src_dev],
                device_id=(my_i,),
                device_id_type=pl.DeviceIdType.MESH,
            )
            recv.wait_recv()

        for rdma in p2_sends:
            rdma.wait_send()

        @functools.partial(
            pl.run_scoped, exit_sem=pltpu.SemaphoreType.REGULAR
        )
        def _(exit_sem):
            for d in range(1, N_DEV):
                peer = (my_i + d) % N_DEV
                pl.semaphore_signal(
                    exit_sem, inc=1,
                    device_id=(peer,), device_id_type=pl.DeviceIdType.MESH,
                )
            pl.semaphore_wait(exit_sem, N_DEV - 1)

    return pl.pallas_call(
        body,
        out_shape=jax.ShapeDtypeStruct((m, n), jnp.float32),
        in_specs=[
            pl.BlockSpec(memory_space=pltpu.VMEM),
            pl.BlockSpec(memory_space=pltpu.VMEM),
        ],
        out_specs=pl.BlockSpec(memory_space=pltpu.VMEM),
        scratch_shapes=[
            pltpu.VMEM((m, n), jnp.bfloat16),
            pltpu.VMEM((N_DEV, SLAB, n), jnp.bfloat16),
            pltpu.SemaphoreType.DMA((N_DEV,)),
            pltpu.SemaphoreType.DMA((N_DEV,)),
            pltpu.SemaphoreType.DMA((N_DEV,)),
            pltpu.SemaphoreType.DMA((N_DEV,)),
        ],
        compiler_params=pltpu.CompilerParams(collective_id=0),
    )(t, W)


# device time: 23953 ns/iter; 1.6253x vs baseline; 1.2353x over previous
import jax
import jax.numpy as jnp
from jax import lax
from jax.experimental import pallas as pl
from jax.experimental.pallas import tpu as pltpu

N_DEV = 32
SLAB = 16


def kernel(t, W):
    m, _ = t.shape
    _, n = W.shape

    def body(t_ref, w_ref, out_ref, stage_ref, p1recv_ref, p2buf_ref,
             p1_send_sems, p1_recv_sems, p2_send_sems, p2_recv_sems):
        my_i = lax.axis_index("i")

        barrier = pltpu.get_barrier_semaphore()
        for d in range(1, N_DEV):
            peer = (my_i + d) % N_DEV
            pl.semaphore_signal(
                barrier, inc=1,
                device_id=(peer,), device_id_type=pl.DeviceIdType.MESH,
            )
        stage_ref[...] = t_ref[...].astype(jnp.bfloat16)
        pl.semaphore_wait(barrier, N_DEV - 1)

        p1_sends = []
        for d in range(1, N_DEV):
            tgt = (my_i + d) % N_DEV
            rdma = pltpu.make_async_remote_copy(
                src_ref=stage_ref.at[pl.ds(tgt * SLAB, SLAB)],
                dst_ref=p1recv_ref.at[my_i],
                send_sem=p1_send_sems.at[tgt],
                recv_sem=p1_recv_sems.at[my_i],
                device_id=(tgt,),
                device_id_type=pl.DeviceIdType.MESH,
            )
            rdma.start()
            p1_sends.append(rdma)

        p1recv_ref[pl.ds(my_i, 1)] = stage_ref[pl.ds(my_i * SLAB, SLAB), :][None]
        for d in range(1, N_DEV):
            src_dev = (my_i + d) % N_DEV
            recv = pltpu.make_async_remote_copy(
                src_ref=stage_ref.at[pl.ds(0, SLAB)],
                dst_ref=p1recv_ref.at[src_dev],
                send_sem=p1_send_sems.at[my_i],
                recv_sem=p1_recv_sems.at[src_dev],
                device_id=(my_i,),
                device_id_type=pl.DeviceIdType.MESH,
            )
            recv.wait_recv()

        ssum = jnp.sum(p1recv_ref[...].astype(jnp.float32), axis=0)
        res = jnp.dot(
            ssum.astype(jnp.bfloat16),
            w_ref[...].astype(jnp.bfloat16),
            preferred_element_type=jnp.float32,
        )
        p2buf_ref[pl.ds(my_i, 1)] = res.astype(jnp.bfloat16)[None]

        for rdma in p1_sends:
            rdma.wait_send()

        p2_sends = []
        for d in range(1, N_DEV):
            tgt = (my_i + d) % N_DEV
            rdma = pltpu.make_async_remote_copy(
                src_ref=p2buf_ref.at[my_i],
                dst_ref=p2buf_ref.at[my_i],
                send_sem=p2_send_sems.at[tgt],
                recv_sem=p2_recv_sems.at[my_i],
                device_id=(tgt,),
                device_id_type=pl.DeviceIdType.MESH,
            )
            rdma.start()
            p2_sends.append(rdma)

        for d in range(1, N_DEV):
            src_dev = (my_i + d) % N_DEV
            recv = pltpu.make_async_remote_copy(
                src_ref=p2buf_ref.at[0],
                dst_ref=p2buf_ref.at[src_dev],
                send_sem=p2_send_sems.at[my_i],
                recv_sem=p2_recv_sems.at[src_dev],
                device_id=(my_i,),
                device_id_type=pl.DeviceIdType.MESH,
            )
            recv.wait_recv()

        out_ref[...] = p2buf_ref[...].reshape(m, n).astype(jnp.float32)

        for rdma in p2_sends:
            rdma.wait_send()

    return pl.pallas_call(
        body,
        out_shape=jax.ShapeDtypeStruct((m, n), jnp.float32),
        in_specs=[
            pl.BlockSpec(memory_space=pltpu.VMEM),
            pl.BlockSpec(memory_space=pltpu.VMEM),
        ],
        out_specs=pl.BlockSpec(memory_space=pltpu.VMEM),
        scratch_shapes=[
            pltpu.VMEM((m, n), jnp.bfloat16),
            pltpu.VMEM((N_DEV, SLAB, n), jnp.bfloat16),
            pltpu.VMEM((N_DEV, SLAB, n), jnp.bfloat16),
            pltpu.SemaphoreType.DMA((N_DEV,)),
            pltpu.SemaphoreType.DMA((N_DEV,)),
            pltpu.SemaphoreType.DMA((N_DEV,)),
            pltpu.SemaphoreType.DMA((N_DEV,)),
        ],
        compiler_params=pltpu.CompilerParams(collective_id=0),
    )(t, W)
